# Initial kernel scaffold; baseline (speedup 1.0000x reference)
import jax
import jax.numpy as jnp
from jax.experimental import pallas as pl


def kernel(x, edge_index, edge_attr, W, b):
    raise NotImplementedError("write your pallas kernel here")


# speedup vs baseline: 23.3555x; 23.3555x over previous
import dataclasses
import functools

import jax
import jax.numpy as jnp
from jax import lax
from jax.experimental import pallas as pl
from jax.experimental.pallas import tpu as pltpu
from jax.experimental.pallas import tpu_sc as plsc

N = 10000
NP = 10240
E = 320000
D = 128
DH = 64
C = 40
ALPHA = 0.05
NC, NS, L = 2, 16, 16

KE = 80
JD = (E // (NC * NS)) // KE
JC = (E // NS) // KE

_mesh = plsc.VectorSubcoreMesh(
    core_axis_name="c", subcore_axis_name="s", num_cores=NC)
_sc_params = pltpu.CompilerParams()
if "needs_layout_passes" in pltpu.CompilerParams.__dataclass_fields__:
    _sc_params = dataclasses.replace(_sc_params, needs_layout_passes=False)
if "use_tc_tiling_on_sc" in pltpu.CompilerParams.__dataclass_fields__:
    _sc_params = dataclasses.replace(_sc_params, use_tc_tiling_on_sc=False)
f32 = jnp.float32
i32 = jnp.int32


def _zero_vec16():
    return jnp.zeros((L,), f32)


@functools.partial(
    pl.kernel,
    out_type=jax.ShapeDtypeStruct((NS, JC, KE), f32),
    mesh=_mesh,
    compiler_params=_sc_params,
    scratch_types=[
        pltpu.VMEM((JC, KE), i32),
        pltpu.VMEM((JC, KE), i32),
        pltpu.VMEM((JC, KE), f32),
        pltpu.VMEM((NP,), f32),
        pltpu.VMEM((NP // NS,), f32),
        pltpu.VMEM_SHARED((NP,), f32),
        pltpu.SemaphoreType.DMA,
    ],
)
def _front_kernel(src_hbm, dst_hbm, attr_hbm, norm_hbm,
                  src_v, dst_v, attr_v, dinv_v, degb, deg_sh, dsem):
    c = lax.axis_index("c")
    s = lax.axis_index("s")
    rows_per_tile = NP // NS
    pltpu.sync_copy(dst_hbm.at[s], dst_v)
    pltpu.sync_copy(attr_hbm.at[s], attr_v)
    pltpu.sync_copy(src_hbm.at[s], src_v)

    @pl.loop(0, rows_per_tile, step=L)
    def _(i):
        degb[pl.ds(i, L)] = _zero_vec16()

    pltpu.sync_copy(degb, deg_sh.at[pl.ds(s * rows_per_tile, rows_per_tile)])
    plsc.subcore_barrier()

    @pl.loop(0, (JC + 7) // 8)
    def _(i):
        j8 = i * 8
        for b in range(8):
            @pl.when(j8 + b < JC)
            def _():
                pltpu.async_copy(attr_v.at[j8 + b],
                                 deg_sh.at[dst_v.at[j8 + b]], dsem, add=True)
        for b in range(8):
            @pl.when(j8 + b < JC)
            def _():
                pltpu.make_async_copy(attr_v.at[j8 + b],
                                      deg_sh.at[dst_v.at[j8 + b]],
                                      dsem).wait()

    plsc.subcore_barrier()
    pltpu.sync_copy(deg_sh.at[pl.ds(s * rows_per_tile, rows_per_tile)], degb)

    @pl.loop(0, rows_per_tile, step=L)
    def _(i):
        d = degb[pl.ds(i, L)]
        y = plsc.bitcast(0x5F3759DF - (plsc.bitcast(d, i32) >> 1), f32)
        for _ in range(3):
            y = y * (1.5 - 0.5 * d * y * y)
        degb[pl.ds(i, L)] = jnp.where(d > 0.0, y, 0.0)

    pltpu.sync_copy(degb, deg_sh.at[pl.ds(s * rows_per_tile, rows_per_tile)])
    plsc.subcore_barrier()
    pltpu.sync_copy(deg_sh, dinv_v)

    @pl.loop(0, JC)
    def _(j):
        for kk in range(KE // L):
            sl = pl.ds(kk * L, L)
            a = plsc.load_gather(dinv_v, [src_v[j, sl]])
            b = plsc.load_gather(dinv_v, [dst_v[j, sl]])
            attr_v[j, sl] = a * attr_v[j, sl] * b

    @pl.when(c == 0)
    def _():
        pltpu.sync_copy(attr_v, norm_hbm.at[s])


@functools.partial(
    pl.kernel,
    out_type=jax.ShapeDtypeStruct((NC, NP, DH), f32),
    mesh=_mesh,
    compiler_params=_sc_params,
    scratch_types=[
        pltpu.VMEM((JC, KE), i32),
        pltpu.VMEM((JC, KE), i32),
        pltpu.VMEM((JC, KE), f32),
        pltpu.VMEM((3, KE, DH), f32),
        pltpu.VMEM((2, KE, DH), f32),
        pltpu.VMEM_SHARED((NP, DH), f32),
    ] + [pltpu.SemaphoreType.DMA] * 5,
)
def _conv_kernel(table_hbm, src_hbm, dst_hbm, norm_hbm, out_hbm,
                 src_v, dst_v, norm_v, rin, rout, acc_sh,
                 g0, g1, g2, s0, s1):
    c = lax.axis_index("c")
    s = lax.axis_index("s")
    rows_per_tile = NP // NS
    gsem = (g0, g1, g2)
    ssem = (s0, s1)
    NBUF = 3
    ROT = 6
    table = table_hbm.at[c]
    pltpu.sync_copy(src_hbm.at[s], src_v)
    pltpu.sync_copy(dst_hbm.at[s], dst_v)
    pltpu.sync_copy(norm_hbm.at[s], norm_v)

    @pl.loop(0, KE)
    def _(r):
        for q in range(DH // L):
            rout[0, r, pl.ds(q * L, L)] = _zero_vec16()

    for i in range(rows_per_tile // KE):
        pltpu.sync_copy(rout.at[0],
                        acc_sh.at[pl.ds(s * rows_per_tile + i * KE, KE)])
    plsc.subcore_barrier()

    def _scale(j, gb, sb):
        @pl.loop(0, KE, step=L)
        def _(k0):
            nv = norm_v[j, pl.ds(k0, L)]
            for t in range(L):
                sc = nv[t]
                for q in range(DH // L):
                    sl = pl.ds(q * L, L)
                    rout[sb, k0 + t, sl] = rin[gb, k0 + t, sl] * sc

    for b in range(NBUF):
        pltpu.async_copy(table.at[src_v.at[b]], rin.at[b], gsem[b])

    @pl.loop(0, (JC + ROT - 1) // ROT)
    def _(i):
        j = i * ROT
        for b in range(ROT):
            jb = j + b
            gb = b % NBUF
            sb = b % 2

            @pl.when(jb < JC)
            def _():
                pltpu.make_async_copy(table.at[src_v.at[jb]], rin.at[gb],
                                      gsem[gb]).wait()

                @pl.when(jb >= 2)
                def _():
                    pltpu.make_async_copy(rout.at[sb],
                                          acc_sh.at[dst_v.at[jb]],
                                          ssem[sb]).wait()

                _scale(jb, gb, sb)

                @pl.when(jb + NBUF < JC)
                def _():
                    pltpu.async_copy(table.at[src_v.at[jb + NBUF]],
                                     rin.at[gb], gsem[gb])

                pltpu.async_copy(rout.at[sb], acc_sh.at[dst_v.at[jb]],
                                 ssem[sb], add=True)

    for sb in range(2):
        pltpu.make_async_copy(rout.at[sb], acc_sh.at[dst_v.at[sb]],
                              ssem[sb]).wait()
    plsc.subcore_barrier()
    pltpu.sync_copy(
        acc_sh.at[pl.ds(s * rows_per_tile, rows_per_tile)],
        out_hbm.at[c, pl.ds(s * rows_per_tile, rows_per_tile)],
    )


_RF = 2000


def _final_body(x_ref, m1_ref, m2_ref, w_ref, b_ref, out_ref):
    c1 = (1.0 - ALPHA) / 2.0
    c2 = c1 * c1
    logits = b_ref[...].astype(f32)
    for h in range(NC):
        a = (c1 * m1_ref[h] + c2 * m2_ref[h] + ALPHA * x_ref[h])
        logits = logits + jax.lax.dot_general(
            a, w_ref[h],
            (((1,), (0,)), ((), ())),
            preferred_element_type=f32,
            precision=jax.lax.Precision.HIGHEST,
        )
    m = jnp.max(logits, axis=1, keepdims=True)
    shifted = logits - m
    lse = jnp.log(jnp.sum(jnp.exp(shifted), axis=1, keepdims=True))
    out_ref[...] = shifted - lse


def _final(x_s, m1, m2, w_s, b):
    blk = lambda: pl.BlockSpec((NC, _RF, DH), lambda i: (0, i, 0))
    return pl.pallas_call(
        _final_body,
        grid=(N // _RF,),
        in_specs=[
            blk(), blk(), blk(),
            pl.BlockSpec((NC, DH, C), lambda i: (0, 0, 0)),
            pl.BlockSpec((1, C), lambda i: (0, 0)),
        ],
        out_specs=pl.BlockSpec((_RF, C), lambda i: (i, 0)),
        out_shape=jax.ShapeDtypeStruct((N, C), f32),
    )(x_s, m1, m2, w_s, b.reshape(1, C))


def kernel(x, edge_index, edge_attr, W, b):
    src = edge_index[0].astype(i32)
    dst = edge_index[1].astype(i32)
    attr = edge_attr.astype(f32)

    srcB = src.reshape(NS, JC, KE)
    dstB = dst.reshape(NS, JC, KE)
    attrB = attr.reshape(NS, JC, KE)

    xp = jnp.pad(x, ((0, NP - N), (0, 0)))
    x_s = jnp.stack([xp[:, :DH], xp[:, DH:]])
    w_s = W.reshape(NC, DH, C)

    normB = _front_kernel(srcB, dstB, attrB)

    m1 = _conv_kernel(x_s, srcB, dstB, normB)
    m2 = _conv_kernel(m1, srcB, dstB, normB)

    return _final(x_s[:, :N, :], m1, m2, w_s, b)
